# baseline (speedup 1.0000x reference)
import jax
import jax.numpy as jnp
from jax import lax
from jax.experimental import pallas as pl
from jax.experimental.pallas import tpu as pltpu

NS = 3
KT = 9
HALF = 4
EPS = 1e-5
VMEM_LIMIT = 64 * 1024 * 1024


def _bn_fold(g, b, m, v):
    s = g / jnp.sqrt(v + EPS)
    return s, b - m * s


def _tapmajor(w):
    co, ci, kt = w.shape
    return jnp.transpose(w, (0, 2, 1)).reshape(co, kt * ci)


def _im2col(src_ref, dst_ref, C, V, TV):
    z = jnp.bfloat16
    for k in range(KT):
        r0 = C * k
        sh = (k - HALF) * V
        if sh < 0:
            dst_ref[pl.ds(r0, C), pl.ds(0, -sh)] = jnp.zeros((C, -sh), z)
            dst_ref[pl.ds(r0, C), pl.ds(-sh, TV + sh)] = src_ref[:, pl.ds(0, TV + sh)]
        elif sh > 0:
            dst_ref[pl.ds(r0, C), pl.ds(0, TV - sh)] = src_ref[:, pl.ds(sh, TV - sh)]
            dst_ref[pl.ds(r0, C), pl.ds(TV - sh, sh)] = jnp.zeros((C, sh), z)
        else:
            dst_ref[pl.ds(r0, C), :] = src_ref[...]


def _make_fused(geom, yoff, pgoff, branches, inv_den):
    C, O, I, T, V, G, TV, L4, r1, r9 = geom
    IG, OG = I * G, O * G
    dims = (((0,), (0,)), ((), ()))

    def softmax_rows(m):
        m = m - jnp.max(m, axis=0, keepdims=True)
        e = jnp.exp(m)
        return e / jnp.sum(e, axis=0, keepdims=True)

    def diag_sum(mf):
        s = mf[0:V, 0:V]
        for j in range(1, 4):
            s = s + mf[j * V:(j + 1) * V, j * V:(j + 1) * V]
        return s * inv_den

    def body(x_ref, w1_ref, b1_ref, w9_ref, b9_ref, acat_ref, c1_ref,
             wt_ref, c3_ref, o_ref, xd_ref, xs_ref, yd_ref, pg_ref, gd_ref,
             gs_ref):
        for t in range(T):
            xd_ref[:, pl.ds(t * V, V)] = x_ref[:, t, :].astype(jnp.bfloat16)
        _im2col(xd_ref, xs_ref, C, V, TV)
        yd_ref[pl.ds(0, r1), :] = (
            jnp.dot(w1_ref[...], xs_ref[pl.ds(HALF * C, C), :],
                    preferred_element_type=jnp.float32) + b1_ref[...]
        ).astype(jnp.bfloat16)
        yd_ref[pl.ds(r1, r9), :] = (
            jnp.dot(w9_ref[...], xs_ref[...],
                    preferred_element_type=jnp.float32) + b9_ref[...]
        ).astype(jnp.bfloat16)

        for name, nrows in branches:
            src, dst = yoff[name], pgoff[name]
            for g in range(G):
                pg_ref[pl.ds(dst + g * nrows, nrows), :] = (
                    yd_ref[pl.ds(src, nrows), pl.ds(g * L4, L4)])

        acc = jnp.zeros((OG, L4), jnp.float32)
        for i in range(NS):
            def ld(name):
                return pg_ref[pl.ds(pgoff[name], IG), :]
            m1 = lax.dot_general(ld(f'a{i}'), ld(f'b{i}'), dims,
                                 preferred_element_type=jnp.float32)
            m2 = lax.dot_general(ld(f't1_{i}'), ld(f't2_{i}'), dims,
                                 preferred_element_type=jnp.float32)
            m3 = lax.dot_general(ld(f's1_{i}'), ld(f's2_{i}'), dims,
                                 preferred_element_type=jnp.float32)
            af = (acat_ref[i] + softmax_rows(diag_sum(m1))
                  + softmax_rows(diag_sum(m2)) + softmax_rows(diag_sum(m3)))
            z = jnp.zeros((V, V), jnp.float32)
            rows = [jnp.concatenate([af if c == j else z for c in range(4)], axis=1)
                    for j in range(4)]
            bmat = jnp.concatenate(rows, axis=0).astype(jnp.bfloat16)
            d = pg_ref[pl.ds(pgoff[f'D{i}'], OG), :]
            acc = acc + jnp.dot(d, bmat, preferred_element_type=jnp.float32)

        for g in range(G):
            dn = yd_ref[pl.ds(yoff['down'], O), pl.ds(g * L4, L4)]
            gd_ref[:, pl.ds(g * L4, L4)] = jnp.maximum(
                acc[g * O:(g + 1) * O, :] + dn.astype(jnp.float32) + c1_ref[...],
                0.0).astype(jnp.bfloat16)

        _im2col(gd_ref, gs_ref, O, V, TV)
        tcn = jnp.dot(wt_ref[...], gs_ref[...], preferred_element_type=jnp.float32)
        res = yd_ref[pl.ds(yoff['res'], O), :].astype(jnp.float32)
        o_ref[...] = jnp.maximum(tcn + res + c3_ref[...], 0.0)
    return body


def kernel(x, A, PA, Wa, ba, Wb, bb, Wt1, bt1, Wt2, bt2, Wd, bd,
           bst1, bst2, Wst1_0, Wst1_1, Wst1_2, Wst2_0, Wst2_1, Wst2_2,
           bn_g, bn_b, bn_m, bn_v, Wdn, bdn, dn_g, dn_b, dn_m, dn_v,
           Wtcn, btcn, tc_g, tc_b, tc_m, tc_v, Wres, bres,
           rs_g, rs_b, rs_m, rs_v):
    N, C, T, V = x.shape
    O = Wd.shape[1]
    I = Wa.shape[1]
    G = T // 4
    TV = T * V
    L4 = 4 * V

    eye = jnp.eye(V, dtype=jnp.float32)[None]
    acat = (4.0 * A ** 2 - A - 2.0 * eye
            + jax.nn.softmax((8.0 * A ** 4 - 4.0 * A ** 2 - 4.0 * A + eye) / V,
                             axis=-2) + PA)

    s_bn, b_bn = _bn_fold(bn_g, bn_b, bn_m, bn_v)
    s_dn, b_dn = _bn_fold(dn_g, dn_b, dn_m, dn_v)
    s_tc, b_tc = _bn_fold(tc_g, tc_b, tc_m, tc_v)
    s_rs, b_rs = _bn_fold(rs_g, rs_b, rs_m, rs_v)
    wd_eff = Wd * s_bn[None, :, None]
    wdn_eff, bdn_eff = Wdn * s_dn[:, None], s_dn * bdn + b_dn
    wres_eff, bres_eff = Wres * s_rs[:, None], s_rs * bres + b_rs
    wtcn_eff = Wtcn * s_tc[:, None, None]
    c1_col = (s_bn * jnp.sum(bd, axis=0) + b_bn)[:, None]
    c3_col = (s_tc * btcn + b_tc)[:, None]

    yoff = {'D0': 0, 'D1': O, 'D2': 2 * O, 'down': 3 * O, 'res': 4 * O}
    r = 5 * O
    w1_rows, b1_rows = [wd_eff[0], wd_eff[1], wd_eff[2], wdn_eff, wres_eff], \
                       [jnp.zeros((3 * O,), jnp.float32), bdn_eff, bres_eff]
    for i in range(NS):
        yoff[f'a{i}'] = r; w1_rows.append(Wa[i]); b1_rows.append(ba[i]); r += I
        yoff[f'b{i}'] = r; w1_rows.append(Wb[i]); b1_rows.append(bb[i]); r += I
    st1 = [Wst1_0, Wst1_1, Wst1_2]
    st2 = [Wst2_0, Wst2_1, Wst2_2]
    for i in (0, 2):
        yoff[f's1_{i}'] = r; w1_rows.append(st1[i]); b1_rows.append(bst1[i]); r += I
        yoff[f's2_{i}'] = r; w1_rows.append(st2[i]); b1_rows.append(bst2[i]); r += I
    r1 = r
    w9_rows, b9_rows = [], []
    for i in range(NS):
        yoff[f't1_{i}'] = r; w9_rows.append(_tapmajor(Wt1[i])); b9_rows.append(bt1[i]); r += I
        yoff[f't2_{i}'] = r; w9_rows.append(_tapmajor(Wt2[i])); b9_rows.append(bt2[i]); r += I
    yoff['s1_1'] = r; w9_rows.append(_tapmajor(st1[1])); b9_rows.append(bst1[1]); r += I
    yoff['s2_1'] = r; w9_rows.append(_tapmajor(st2[1])); b9_rows.append(bst2[1]); r += I
    r9 = r - r1

    attn = [f'{p}{i}' for i in range(NS) for p in
            ('a', 'b', 't1_', 't2_', 's1_', 's2_')]
    pgoff, p = {}, 0
    branches = []
    for name in attn:
        pgoff[name] = p; branches.append((name, I)); p += I * G
    for i in range(NS):
        pgoff[f'D{i}'] = p; branches.append((f'D{i}', O)); p += O * G
    pg_rows = p

    w1 = jnp.concatenate(w1_rows, axis=0).astype(jnp.bfloat16)
    b1 = jnp.concatenate(b1_rows, axis=0)[:, None]
    w9 = jnp.concatenate(w9_rows, axis=0).astype(jnp.bfloat16)
    b9 = jnp.concatenate(b9_rows, axis=0)[:, None]
    wtf = _tapmajor(wtcn_eff).astype(jnp.bfloat16)

    geom = (C, O, I, T, V, G, TV, L4, r1, r9)
    out = pl.pallas_call(
        _make_fused(geom, yoff, pgoff, branches, 1.0 / float(I * T)),
        out_shape=jax.ShapeDtypeStruct((N, O, TV), jnp.float32),
        grid=(N,),
        in_specs=[
            pl.BlockSpec((None, C, T, V), lambda n: (n, 0, 0, 0)),
            pl.BlockSpec(w1.shape, lambda n: (0, 0)),
            pl.BlockSpec(b1.shape, lambda n: (0, 0)),
            pl.BlockSpec(w9.shape, lambda n: (0, 0)),
            pl.BlockSpec(b9.shape, lambda n: (0, 0)),
            pl.BlockSpec(acat.shape, lambda n: (0, 0, 0)),
            pl.BlockSpec(c1_col.shape, lambda n: (0, 0)),
            pl.BlockSpec(wtf.shape, lambda n: (0, 0)),
            pl.BlockSpec(c3_col.shape, lambda n: (0, 0)),
        ],
        out_specs=pl.BlockSpec((None, O, TV), lambda n: (n, 0, 0)),
        scratch_shapes=[
            pltpu.VMEM((C, TV), jnp.bfloat16),
            pltpu.VMEM((KT * C, TV), jnp.bfloat16),
            pltpu.VMEM((r1 + r9, TV), jnp.bfloat16),
            pltpu.VMEM((pg_rows, L4), jnp.bfloat16),
            pltpu.VMEM((O, TV), jnp.bfloat16),
            pltpu.VMEM((KT * O, TV), jnp.bfloat16),
        ],
        compiler_params=pltpu.CompilerParams(dimension_semantics=("parallel",),
                                             vmem_limit_bytes=VMEM_LIMIT),
    )(x, w1, b1, w9, b9, acat, c1_col, wtf, c3_col)
    return out.reshape(N, O, T, V)


# speedup vs baseline: 1.4067x; 1.4067x over previous
import jax
import jax.numpy as jnp
from jax import lax
from jax.experimental import pallas as pl
from jax.experimental.pallas import tpu as pltpu

NS = 3
KT = 9
HALF = 4
EPS = 1e-5
VMEM_LIMIT = 64 * 1024 * 1024


def _bn_fold(g, b, m, v):
    s = g / jnp.sqrt(v + EPS)
    return s, b - m * s


def _tapmajor(w):
    co, ci, kt = w.shape
    return jnp.transpose(w, (0, 2, 1)).reshape(co, kt * ci)


def _im2col(src_ref, dst_ref, C, V, TV):
    z = jnp.bfloat16
    for k in range(KT):
        r0 = C * k
        sh = (k - HALF) * V
        if sh < 0:
            dst_ref[pl.ds(r0, C), pl.ds(0, -sh)] = jnp.zeros((C, -sh), z)
            dst_ref[pl.ds(r0, C), pl.ds(-sh, TV + sh)] = src_ref[:, pl.ds(0, TV + sh)]
        elif sh > 0:
            dst_ref[pl.ds(r0, C), pl.ds(0, TV - sh)] = src_ref[:, pl.ds(sh, TV - sh)]
            dst_ref[pl.ds(r0, C), pl.ds(TV - sh, sh)] = jnp.zeros((C, sh), z)
        else:
            dst_ref[pl.ds(r0, C), :] = src_ref[...]


def _make_fused(geom, yoff, pgoff, branches, inv_den):
    C, O, I, T, V, G, TV, L4, r1, r9 = geom
    IG, OG = I * G, O * G
    dims = (((0,), (0,)), ((), ()))

    def softmax_rows(m):
        m = m - jnp.max(m, axis=0, keepdims=True)
        e = jnp.exp(m)
        return e / jnp.sum(e, axis=0, keepdims=True)

    def diag_sum(mf):
        s = mf[0:V, 0:V]
        for j in range(1, 4):
            s = s + mf[j * V:(j + 1) * V, j * V:(j + 1) * V]
        return s * inv_den

    def body(x_ref, w1_ref, b1_ref, w9_ref, b9_ref, acat_ref, c1_ref,
             wt_ref, c3_ref, o_ref, xs_ref, yd_ref, pg_ref, gd_ref, gs_ref):
        _im2col(x_ref, xs_ref, C, V, TV)
        yd_ref[pl.ds(0, r1), :] = (
            jnp.dot(w1_ref[...], xs_ref[pl.ds(HALF * C, C), :],
                    preferred_element_type=jnp.float32) + b1_ref[...]
        ).astype(jnp.bfloat16)
        yd_ref[pl.ds(r1, r9), :] = (
            jnp.dot(w9_ref[...], xs_ref[...],
                    preferred_element_type=jnp.float32) + b9_ref[...]
        ).astype(jnp.bfloat16)

        for name, nrows in branches:
            src, dst = yoff[name], pgoff[name]
            for g in range(G):
                pg_ref[pl.ds(dst + g * nrows, nrows), :] = (
                    yd_ref[pl.ds(src, nrows), pl.ds(g * L4, L4)])

        acc = jnp.zeros((OG, L4), jnp.float32)
        for i in range(NS):
            def ld(name):
                return pg_ref[pl.ds(pgoff[name], IG), :]
            m1 = lax.dot_general(ld(f'a{i}'), ld(f'b{i}'), dims,
                                 preferred_element_type=jnp.float32)
            m2 = lax.dot_general(ld(f't1_{i}'), ld(f't2_{i}'), dims,
                                 preferred_element_type=jnp.float32)
            m3 = lax.dot_general(ld(f's1_{i}'), ld(f's2_{i}'), dims,
                                 preferred_element_type=jnp.float32)
            af = (acat_ref[i] + softmax_rows(diag_sum(m1))
                  + softmax_rows(diag_sum(m2)) + softmax_rows(diag_sum(m3)))
            z = jnp.zeros((V, V), jnp.float32)
            rows = [jnp.concatenate([af if c == j else z for c in range(4)], axis=1)
                    for j in range(4)]
            bmat = jnp.concatenate(rows, axis=0).astype(jnp.bfloat16)
            d = pg_ref[pl.ds(pgoff[f'D{i}'], OG), :]
            acc = acc + jnp.dot(d, bmat, preferred_element_type=jnp.float32)

        for g in range(G):
            dn = yd_ref[pl.ds(yoff['down'], O), pl.ds(g * L4, L4)]
            gd_ref[:, pl.ds(g * L4, L4)] = jnp.maximum(
                acc[g * O:(g + 1) * O, :] + dn.astype(jnp.float32) + c1_ref[...],
                0.0).astype(jnp.bfloat16)

        _im2col(gd_ref, gs_ref, O, V, TV)
        tcn = jnp.dot(wt_ref[...], gs_ref[...], preferred_element_type=jnp.float32)
        res = yd_ref[pl.ds(yoff['res'], O), :].astype(jnp.float32)
        o_ref[...] = jnp.maximum(tcn + res + c3_ref[...], 0.0)
    return body


def kernel(x, A, PA, Wa, ba, Wb, bb, Wt1, bt1, Wt2, bt2, Wd, bd,
           bst1, bst2, Wst1_0, Wst1_1, Wst1_2, Wst2_0, Wst2_1, Wst2_2,
           bn_g, bn_b, bn_m, bn_v, Wdn, bdn, dn_g, dn_b, dn_m, dn_v,
           Wtcn, btcn, tc_g, tc_b, tc_m, tc_v, Wres, bres,
           rs_g, rs_b, rs_m, rs_v):
    N, C, T, V = x.shape
    O = Wd.shape[1]
    I = Wa.shape[1]
    G = T // 4
    TV = T * V
    L4 = 4 * V

    eye = jnp.eye(V, dtype=jnp.float32)[None]
    acat = (4.0 * A ** 2 - A - 2.0 * eye
            + jax.nn.softmax((8.0 * A ** 4 - 4.0 * A ** 2 - 4.0 * A + eye) / V,
                             axis=-2) + PA)

    s_bn, b_bn = _bn_fold(bn_g, bn_b, bn_m, bn_v)
    s_dn, b_dn = _bn_fold(dn_g, dn_b, dn_m, dn_v)
    s_tc, b_tc = _bn_fold(tc_g, tc_b, tc_m, tc_v)
    s_rs, b_rs = _bn_fold(rs_g, rs_b, rs_m, rs_v)
    wd_eff = Wd * s_bn[None, :, None]
    wdn_eff, bdn_eff = Wdn * s_dn[:, None], s_dn * bdn + b_dn
    wres_eff, bres_eff = Wres * s_rs[:, None], s_rs * bres + b_rs
    wtcn_eff = Wtcn * s_tc[:, None, None]
    c1_col = (s_bn * jnp.sum(bd, axis=0) + b_bn)[:, None]
    c3_col = (s_tc * btcn + b_tc)[:, None]

    yoff = {'D0': 0, 'D1': O, 'D2': 2 * O, 'down': 3 * O, 'res': 4 * O}
    r = 5 * O
    w1_rows, b1_rows = [wd_eff[0], wd_eff[1], wd_eff[2], wdn_eff, wres_eff], \
                       [jnp.zeros((3 * O,), jnp.float32), bdn_eff, bres_eff]
    for i in range(NS):
        yoff[f'a{i}'] = r; w1_rows.append(Wa[i]); b1_rows.append(ba[i]); r += I
        yoff[f'b{i}'] = r; w1_rows.append(Wb[i]); b1_rows.append(bb[i]); r += I
    st1 = [Wst1_0, Wst1_1, Wst1_2]
    st2 = [Wst2_0, Wst2_1, Wst2_2]
    for i in (0, 2):
        yoff[f's1_{i}'] = r; w1_rows.append(st1[i]); b1_rows.append(bst1[i]); r += I
        yoff[f's2_{i}'] = r; w1_rows.append(st2[i]); b1_rows.append(bst2[i]); r += I
    r1 = r
    w9_rows, b9_rows = [], []
    for i in range(NS):
        yoff[f't1_{i}'] = r; w9_rows.append(_tapmajor(Wt1[i])); b9_rows.append(bt1[i]); r += I
        yoff[f't2_{i}'] = r; w9_rows.append(_tapmajor(Wt2[i])); b9_rows.append(bt2[i]); r += I
    yoff['s1_1'] = r; w9_rows.append(_tapmajor(st1[1])); b9_rows.append(bst1[1]); r += I
    yoff['s2_1'] = r; w9_rows.append(_tapmajor(st2[1])); b9_rows.append(bst2[1]); r += I
    r9 = r - r1

    attn = [f'{p}{i}' for i in range(NS) for p in
            ('a', 'b', 't1_', 't2_', 's1_', 's2_')]
    pgoff, p = {}, 0
    branches = []
    for name in attn:
        pgoff[name] = p; branches.append((name, I)); p += I * G
    for i in range(NS):
        pgoff[f'D{i}'] = p; branches.append((f'D{i}', O)); p += O * G
    pg_rows = p

    w1 = jnp.concatenate(w1_rows, axis=0).astype(jnp.bfloat16)
    b1 = jnp.concatenate(b1_rows, axis=0)[:, None]
    w9 = jnp.concatenate(w9_rows, axis=0).astype(jnp.bfloat16)
    b9 = jnp.concatenate(b9_rows, axis=0)[:, None]
    wtf = _tapmajor(wtcn_eff).astype(jnp.bfloat16)

    geom = (C, O, I, T, V, G, TV, L4, r1, r9)
    xb = x.reshape(N, C, TV).astype(jnp.bfloat16)
    out = pl.pallas_call(
        _make_fused(geom, yoff, pgoff, branches, 1.0 / float(I * T)),
        out_shape=jax.ShapeDtypeStruct((N, O, TV), jnp.float32),
        grid=(N,),
        in_specs=[
            pl.BlockSpec((None, C, TV), lambda n: (n, 0, 0)),
            pl.BlockSpec(w1.shape, lambda n: (0, 0)),
            pl.BlockSpec(b1.shape, lambda n: (0, 0)),
            pl.BlockSpec(w9.shape, lambda n: (0, 0)),
            pl.BlockSpec(b9.shape, lambda n: (0, 0)),
            pl.BlockSpec(acat.shape, lambda n: (0, 0, 0)),
            pl.BlockSpec(c1_col.shape, lambda n: (0, 0)),
            pl.BlockSpec(wtf.shape, lambda n: (0, 0)),
            pl.BlockSpec(c3_col.shape, lambda n: (0, 0)),
        ],
        out_specs=pl.BlockSpec((None, O, TV), lambda n: (n, 0, 0)),
        scratch_shapes=[
            pltpu.VMEM((KT * C, TV), jnp.bfloat16),
            pltpu.VMEM((r1 + r9, TV), jnp.bfloat16),
            pltpu.VMEM((pg_rows, L4), jnp.bfloat16),
            pltpu.VMEM((O, TV), jnp.bfloat16),
            pltpu.VMEM((KT * O, TV), jnp.bfloat16),
        ],
        compiler_params=pltpu.CompilerParams(dimension_semantics=("parallel",),
                                             vmem_limit_bytes=VMEM_LIMIT),
    )(xb, w1, b1, w9, b9, acat, c1_col, wtf, c3_col)
    return out.reshape(N, O, T, V)


# speedup vs baseline: 1.5522x; 1.1034x over previous
import jax
import jax.numpy as jnp
from jax import lax
from jax.experimental import pallas as pl
from jax.experimental.pallas import tpu as pltpu

NS = 3
KT = 9
HALF = 4
EPS = 1e-5
VMEM_LIMIT = 64 * 1024 * 1024


def _bn_fold(g, b, m, v):
    s = g / jnp.sqrt(v + EPS)
    return s, b - m * s


def _tapmajor(w):
    co, ci, kt = w.shape
    return jnp.transpose(w, (0, 2, 1)).reshape(co, kt * ci)


def _im2col(src_ref, dst_ref, C, V, TV):
    z = jnp.bfloat16
    for k in range(KT):
        r0 = C * k
        sh = (k - HALF) * V
        if sh < 0:
            dst_ref[pl.ds(r0, C), pl.ds(0, -sh)] = jnp.zeros((C, -sh), z)
            dst_ref[pl.ds(r0, C), pl.ds(-sh, TV + sh)] = src_ref[:, pl.ds(0, TV + sh)]
        elif sh > 0:
            dst_ref[pl.ds(r0, C), pl.ds(0, TV - sh)] = src_ref[:, pl.ds(sh, TV - sh)]
            dst_ref[pl.ds(r0, C), pl.ds(TV - sh, sh)] = jnp.zeros((C, sh), z)
        else:
            dst_ref[pl.ds(r0, C), :] = src_ref[...]


def _make_fused(geom, yoff, pgoff, branches, inv_den):
    C, O, I, T, V, G, TV, L4, r1, r9 = geom
    IG, OG = I * G, O * G
    dims = (((0,), (0,)), ((), ()))

    def softmax_rows(m):
        m = m - jnp.max(m, axis=0, keepdims=True)
        e = jnp.exp(m)
        return e / jnp.sum(e, axis=0, keepdims=True)

    def diag_sum(mf):
        s = mf[0:V, 0:V]
        for j in range(1, 4):
            s = s + mf[j * V:(j + 1) * V, j * V:(j + 1) * V]
        return s * inv_den

    def body(x_ref, w1_ref, b1_ref, w9_ref, b9_ref, acat_ref, c1_ref,
             wt_ref, c3_ref, o_ref, xs_ref, yd_ref, pg_ref, gd_ref, gs_ref):
        _im2col(x_ref, xs_ref, C, V, TV)
        yd_ref[pl.ds(0, r1), :] = (
            jnp.dot(w1_ref[...], xs_ref[pl.ds(HALF * C, C), :],
                    preferred_element_type=jnp.float32) + b1_ref[...]
        ).astype(jnp.bfloat16)
        yd_ref[pl.ds(r1, r9), :] = (
            jnp.dot(w9_ref[...], xs_ref[...],
                    preferred_element_type=jnp.float32) + b9_ref[...]
        ).astype(jnp.bfloat16)

        for name, nrows in branches:
            src, dst = yoff[name], pgoff[name]
            for g in range(G):
                pg_ref[pl.ds(dst + g * nrows, nrows), :] = (
                    yd_ref[pl.ds(src, nrows), pl.ds(g * L4, L4)])

        acc = c1_ref[...] + pg_ref[pl.ds(pgoff['down'], OG), :].astype(jnp.float32)
        for i in range(NS):
            def ld(name):
                return pg_ref[pl.ds(pgoff[name], IG), :]
            m1 = lax.dot_general(ld(f'a{i}'), ld(f'b{i}'), dims,
                                 preferred_element_type=jnp.float32)
            m2 = lax.dot_general(ld(f't1_{i}'), ld(f't2_{i}'), dims,
                                 preferred_element_type=jnp.float32)
            m3 = lax.dot_general(ld(f's1_{i}'), ld(f's2_{i}'), dims,
                                 preferred_element_type=jnp.float32)
            af = (acat_ref[i] + softmax_rows(diag_sum(m1))
                  + softmax_rows(diag_sum(m2)) + softmax_rows(diag_sum(m3)))
            z = jnp.zeros((V, V), jnp.float32)
            rows = [jnp.concatenate([af if c == j else z for c in range(4)], axis=1)
                    for j in range(4)]
            bmat = jnp.concatenate(rows, axis=0).astype(jnp.bfloat16)
            d = pg_ref[pl.ds(pgoff[f'D{i}'], OG), :]
            acc = acc + jnp.dot(d, bmat, preferred_element_type=jnp.float32)
        acc = jnp.maximum(acc, 0.0)

        for g in range(G):
            gd_ref[:, pl.ds(g * L4, L4)] = (
                acc[g * O:(g + 1) * O, :].astype(jnp.bfloat16))

        _im2col(gd_ref, gs_ref, O, V, TV)
        tcn = jnp.dot(wt_ref[...], gs_ref[...], preferred_element_type=jnp.float32)
        res = yd_ref[pl.ds(yoff['res'], O), :].astype(jnp.float32)
        o_ref[...] = jnp.maximum(tcn + res + c3_ref[...], 0.0)
    return body


def kernel(x, A, PA, Wa, ba, Wb, bb, Wt1, bt1, Wt2, bt2, Wd, bd,
           bst1, bst2, Wst1_0, Wst1_1, Wst1_2, Wst2_0, Wst2_1, Wst2_2,
           bn_g, bn_b, bn_m, bn_v, Wdn, bdn, dn_g, dn_b, dn_m, dn_v,
           Wtcn, btcn, tc_g, tc_b, tc_m, tc_v, Wres, bres,
           rs_g, rs_b, rs_m, rs_v):
    N, C, T, V = x.shape
    O = Wd.shape[1]
    I = Wa.shape[1]
    G = T // 4
    TV = T * V
    L4 = 4 * V

    eye = jnp.eye(V, dtype=jnp.float32)[None]
    acat = (4.0 * A ** 2 - A - 2.0 * eye
            + jax.nn.softmax((8.0 * A ** 4 - 4.0 * A ** 2 - 4.0 * A + eye) / V,
                             axis=-2) + PA)

    s_bn, b_bn = _bn_fold(bn_g, bn_b, bn_m, bn_v)
    s_dn, b_dn = _bn_fold(dn_g, dn_b, dn_m, dn_v)
    s_tc, b_tc = _bn_fold(tc_g, tc_b, tc_m, tc_v)
    s_rs, b_rs = _bn_fold(rs_g, rs_b, rs_m, rs_v)
    wd_eff = Wd * s_bn[None, :, None]
    wdn_eff, bdn_eff = Wdn * s_dn[:, None], s_dn * bdn + b_dn
    wres_eff, bres_eff = Wres * s_rs[:, None], s_rs * bres + b_rs
    wtcn_eff = Wtcn * s_tc[:, None, None]
    c1 = s_bn * jnp.sum(bd, axis=0) + b_bn
    c1_col = jnp.tile(c1, (G,))[:, None]
    c3_col = (s_tc * btcn + b_tc)[:, None]

    yoff = {'D0': 0, 'D1': O, 'D2': 2 * O, 'down': 3 * O, 'res': 4 * O}
    r = 5 * O
    w1_rows, b1_rows = [wd_eff[0], wd_eff[1], wd_eff[2], wdn_eff, wres_eff], \
                       [jnp.zeros((3 * O,), jnp.float32), bdn_eff, bres_eff]
    for i in range(NS):
        yoff[f'a{i}'] = r; w1_rows.append(Wa[i]); b1_rows.append(ba[i]); r += I
        yoff[f'b{i}'] = r; w1_rows.append(Wb[i]); b1_rows.append(bb[i]); r += I
    st1 = [Wst1_0, Wst1_1, Wst1_2]
    st2 = [Wst2_0, Wst2_1, Wst2_2]
    for i in (0, 2):
        yoff[f's1_{i}'] = r; w1_rows.append(st1[i]); b1_rows.append(bst1[i]); r += I
        yoff[f's2_{i}'] = r; w1_rows.append(st2[i]); b1_rows.append(bst2[i]); r += I
    r1 = r
    w9_rows, b9_rows = [], []
    for i in range(NS):
        yoff[f't1_{i}'] = r; w9_rows.append(_tapmajor(Wt1[i])); b9_rows.append(bt1[i]); r += I
        yoff[f't2_{i}'] = r; w9_rows.append(_tapmajor(Wt2[i])); b9_rows.append(bt2[i]); r += I
    yoff['s1_1'] = r; w9_rows.append(_tapmajor(st1[1])); b9_rows.append(bst1[1]); r += I
    yoff['s2_1'] = r; w9_rows.append(_tapmajor(st2[1])); b9_rows.append(bst2[1]); r += I
    r9 = r - r1

    attn = [f'{p}{i}' for i in range(NS) for p in
            ('a', 'b', 't1_', 't2_', 's1_', 's2_')]
    pgoff, p = {}, 0
    branches = []
    for name in attn:
        pgoff[name] = p; branches.append((name, I)); p += I * G
    for i in range(NS):
        pgoff[f'D{i}'] = p; branches.append((f'D{i}', O)); p += O * G
    pgoff['down'] = p; branches.append(('down', O)); p += O * G
    pg_rows = p

    w1 = jnp.concatenate(w1_rows, axis=0).astype(jnp.bfloat16)
    b1 = jnp.concatenate(b1_rows, axis=0)[:, None]
    w9 = jnp.concatenate(w9_rows, axis=0).astype(jnp.bfloat16)
    b9 = jnp.concatenate(b9_rows, axis=0)[:, None]
    wtf = _tapmajor(wtcn_eff).astype(jnp.bfloat16)

    geom = (C, O, I, T, V, G, TV, L4, r1, r9)
    xb = x.reshape(N, C, TV).astype(jnp.bfloat16)
    out = pl.pallas_call(
        _make_fused(geom, yoff, pgoff, branches, 1.0 / float(I * T)),
        out_shape=jax.ShapeDtypeStruct((N, O, TV), jnp.float32),
        grid=(N,),
        in_specs=[
            pl.BlockSpec((None, C, TV), lambda n: (n, 0, 0)),
            pl.BlockSpec(w1.shape, lambda n: (0, 0)),
            pl.BlockSpec(b1.shape, lambda n: (0, 0)),
            pl.BlockSpec(w9.shape, lambda n: (0, 0)),
            pl.BlockSpec(b9.shape, lambda n: (0, 0)),
            pl.BlockSpec(acat.shape, lambda n: (0, 0, 0)),
            pl.BlockSpec(c1_col.shape, lambda n: (0, 0)),
            pl.BlockSpec(wtf.shape, lambda n: (0, 0)),
            pl.BlockSpec(c3_col.shape, lambda n: (0, 0)),
        ],
        out_specs=pl.BlockSpec((None, O, TV), lambda n: (n, 0, 0)),
        scratch_shapes=[
            pltpu.VMEM((KT * C, TV), jnp.bfloat16),
            pltpu.VMEM((r1 + r9, TV), jnp.bfloat16),
            pltpu.VMEM((pg_rows, L4), jnp.bfloat16),
            pltpu.VMEM((O, TV), jnp.bfloat16),
            pltpu.VMEM((KT * O, TV), jnp.bfloat16),
        ],
        compiler_params=pltpu.CompilerParams(dimension_semantics=("parallel",),
                                             vmem_limit_bytes=VMEM_LIMIT),
    )(xb, w1, b1, w9, b9, acat, c1_col, wtf, c3_col)
    return out.reshape(N, O, T, V)


# speedup vs baseline: 1.6371x; 1.0547x over previous
import jax
import jax.numpy as jnp
from jax import lax
from jax.experimental import pallas as pl
from jax.experimental.pallas import tpu as pltpu

NS = 3
KT = 9
HALF = 4
EPS = 1e-5
VMEM_LIMIT = 64 * 1024 * 1024


def _bn_fold(g, b, m, v):
    s = g / jnp.sqrt(v + EPS)
    return s, b - m * s


def _tapmajor(w):
    co, ci, kt = w.shape
    return jnp.transpose(w, (0, 2, 1)).reshape(co, kt * ci)


def _im2col(src_ref, dst_ref, C, V, TV):
    z = jnp.bfloat16
    for k in range(KT):
        r0 = C * k
        sh = (k - HALF) * V
        if sh < 0:
            dst_ref[pl.ds(r0, C), pl.ds(0, -sh)] = jnp.zeros((C, -sh), z)
            dst_ref[pl.ds(r0, C), pl.ds(-sh, TV + sh)] = src_ref[:, pl.ds(0, TV + sh)]
        elif sh > 0:
            dst_ref[pl.ds(r0, C), pl.ds(0, TV - sh)] = src_ref[:, pl.ds(sh, TV - sh)]
            dst_ref[pl.ds(r0, C), pl.ds(TV - sh, sh)] = jnp.zeros((C, sh), z)
        else:
            dst_ref[pl.ds(r0, C), :] = src_ref[...]


def _make_fused(geom, yoff, pgoff, branches, inv_den):
    C, O, I, T, V, G, TV, L4, r1, r9 = geom
    IG, OG = I * G, O * G
    dims = (((0,), (0,)), ((), ()))

    def softmax_rows(m):
        m = m - jnp.max(m, axis=0, keepdims=True)
        e = jnp.exp(m)
        return e / jnp.sum(e, axis=0, keepdims=True)

    def diag_sum(mf):
        s = mf[0:V, 0:V]
        for j in range(1, 4):
            s = s + mf[j * V:(j + 1) * V, j * V:(j + 1) * V]
        return s * inv_den

    def body(x_ref, w1_ref, b1_ref, w9_ref, b9_ref, acat_ref, c1_ref,
             wt_ref, c3_ref, o_ref, xs_ref, yd_ref, pg_ref, pgd_ref, gd_ref,
             gs_ref):
        _im2col(x_ref, xs_ref, C, V, TV)
        yd_ref[pl.ds(0, r1), :] = (
            jnp.dot(w1_ref[...], xs_ref[pl.ds(HALF * C, C), :],
                    preferred_element_type=jnp.float32) + b1_ref[...]
        ).astype(jnp.bfloat16)
        yd_ref[pl.ds(r1, r9), :] = (
            jnp.dot(w9_ref[...], xs_ref[...],
                    preferred_element_type=jnp.float32) + b9_ref[...]
        ).astype(jnp.bfloat16)

        for name, nrows in branches:
            src, dst = yoff[name], pgoff[name]
            for g in range(G):
                pg_ref[pl.ds(dst + g * nrows, nrows), :] = (
                    yd_ref[pl.ds(src, nrows), pl.ds(g * L4, L4)])
        for i in range(NS):
            src = yoff[f'D{i}']
            for g in range(G):
                pgd_ref[pl.ds(g * O, O), pl.ds(i * L4, L4)] = (
                    yd_ref[pl.ds(src, O), pl.ds(g * L4, L4)])

        acc = c1_ref[...] + pg_ref[pl.ds(pgoff['down'], OG), :].astype(jnp.float32)
        bmats = []
        for i in range(NS):
            def ld(name):
                return pg_ref[pl.ds(pgoff[name], IG), :]
            m1 = lax.dot_general(ld(f'a{i}'), ld(f'b{i}'), dims,
                                 preferred_element_type=jnp.float32)
            m2 = lax.dot_general(ld(f't1_{i}'), ld(f't2_{i}'), dims,
                                 preferred_element_type=jnp.float32)
            m3 = lax.dot_general(ld(f's1_{i}'), ld(f's2_{i}'), dims,
                                 preferred_element_type=jnp.float32)
            af = (acat_ref[i] + softmax_rows(diag_sum(m1))
                  + softmax_rows(diag_sum(m2)) + softmax_rows(diag_sum(m3)))
            z = jnp.zeros((V, V), jnp.float32)
            rows = [jnp.concatenate([af if c == j else z for c in range(4)], axis=1)
                    for j in range(4)]
            bmats.append(jnp.concatenate(rows, axis=0))
        bcat = jnp.concatenate(bmats, axis=0).astype(jnp.bfloat16)
        acc = acc + jnp.dot(pgd_ref[...], bcat, preferred_element_type=jnp.float32)
        acc = jnp.maximum(acc, 0.0)

        for g in range(G):
            gd_ref[:, pl.ds(g * L4, L4)] = (
                acc[g * O:(g + 1) * O, :].astype(jnp.bfloat16))

        _im2col(gd_ref, gs_ref, O, V, TV)
        tcn = jnp.dot(wt_ref[...], gs_ref[...], preferred_element_type=jnp.float32)
        res = yd_ref[pl.ds(yoff['res'], O), :].astype(jnp.float32)
        o_ref[...] = jnp.maximum(tcn + res + c3_ref[...], 0.0)
    return body


def kernel(x, A, PA, Wa, ba, Wb, bb, Wt1, bt1, Wt2, bt2, Wd, bd,
           bst1, bst2, Wst1_0, Wst1_1, Wst1_2, Wst2_0, Wst2_1, Wst2_2,
           bn_g, bn_b, bn_m, bn_v, Wdn, bdn, dn_g, dn_b, dn_m, dn_v,
           Wtcn, btcn, tc_g, tc_b, tc_m, tc_v, Wres, bres,
           rs_g, rs_b, rs_m, rs_v):
    N, C, T, V = x.shape
    O = Wd.shape[1]
    I = Wa.shape[1]
    G = T // 4
    TV = T * V
    L4 = 4 * V

    eye = jnp.eye(V, dtype=jnp.float32)[None]
    acat = (4.0 * A ** 2 - A - 2.0 * eye
            + jax.nn.softmax((8.0 * A ** 4 - 4.0 * A ** 2 - 4.0 * A + eye) / V,
                             axis=-2) + PA)

    s_bn, b_bn = _bn_fold(bn_g, bn_b, bn_m, bn_v)
    s_dn, b_dn = _bn_fold(dn_g, dn_b, dn_m, dn_v)
    s_tc, b_tc = _bn_fold(tc_g, tc_b, tc_m, tc_v)
    s_rs, b_rs = _bn_fold(rs_g, rs_b, rs_m, rs_v)
    wd_eff = Wd * s_bn[None, :, None]
    wdn_eff, bdn_eff = Wdn * s_dn[:, None], s_dn * bdn + b_dn
    wres_eff, bres_eff = Wres * s_rs[:, None], s_rs * bres + b_rs
    wtcn_eff = Wtcn * s_tc[:, None, None]
    c1 = s_bn * jnp.sum(bd, axis=0) + b_bn
    c1_col = jnp.tile(c1, (G,))[:, None]
    c3_col = (s_tc * btcn + b_tc)[:, None]

    yoff = {'D0': 0, 'D1': O, 'D2': 2 * O, 'down': 3 * O, 'res': 4 * O}
    r = 5 * O
    w1_rows, b1_rows = [wd_eff[0], wd_eff[1], wd_eff[2], wdn_eff, wres_eff], \
                       [jnp.zeros((3 * O,), jnp.float32), bdn_eff, bres_eff]
    for i in range(NS):
        yoff[f'a{i}'] = r; w1_rows.append(Wa[i]); b1_rows.append(ba[i]); r += I
        yoff[f'b{i}'] = r; w1_rows.append(Wb[i]); b1_rows.append(bb[i]); r += I
    st1 = [Wst1_0, Wst1_1, Wst1_2]
    st2 = [Wst2_0, Wst2_1, Wst2_2]
    for i in (0, 2):
        yoff[f's1_{i}'] = r; w1_rows.append(st1[i]); b1_rows.append(bst1[i]); r += I
        yoff[f's2_{i}'] = r; w1_rows.append(st2[i]); b1_rows.append(bst2[i]); r += I
    r1 = r
    w9_rows, b9_rows = [], []
    for i in range(NS):
        yoff[f't1_{i}'] = r; w9_rows.append(_tapmajor(Wt1[i])); b9_rows.append(bt1[i]); r += I
        yoff[f't2_{i}'] = r; w9_rows.append(_tapmajor(Wt2[i])); b9_rows.append(bt2[i]); r += I
    yoff['s1_1'] = r; w9_rows.append(_tapmajor(st1[1])); b9_rows.append(bst1[1]); r += I
    yoff['s2_1'] = r; w9_rows.append(_tapmajor(st2[1])); b9_rows.append(bst2[1]); r += I
    r9 = r - r1

    attn = [f'{p}{i}' for i in range(NS) for p in
            ('a', 'b', 't1_', 't2_', 's1_', 's2_')]
    pgoff, p = {}, 0
    branches = []
    for name in attn:
        pgoff[name] = p; branches.append((name, I)); p += I * G
    pgoff['down'] = p; branches.append(('down', O)); p += O * G
    pg_rows = p

    w1 = jnp.concatenate(w1_rows, axis=0).astype(jnp.bfloat16)
    b1 = jnp.concatenate(b1_rows, axis=0)[:, None]
    w9 = jnp.concatenate(w9_rows, axis=0).astype(jnp.bfloat16)
    b9 = jnp.concatenate(b9_rows, axis=0)[:, None]
    wtf = _tapmajor(wtcn_eff).astype(jnp.bfloat16)

    geom = (C, O, I, T, V, G, TV, L4, r1, r9)
    xb = x.reshape(N, C, TV).astype(jnp.bfloat16)
    out = pl.pallas_call(
        _make_fused(geom, yoff, pgoff, branches, 1.0 / float(I * T)),
        out_shape=jax.ShapeDtypeStruct((N, O, TV), jnp.float32),
        grid=(N,),
        in_specs=[
            pl.BlockSpec((None, C, TV), lambda n: (n, 0, 0)),
            pl.BlockSpec(w1.shape, lambda n: (0, 0)),
            pl.BlockSpec(b1.shape, lambda n: (0, 0)),
            pl.BlockSpec(w9.shape, lambda n: (0, 0)),
            pl.BlockSpec(b9.shape, lambda n: (0, 0)),
            pl.BlockSpec(acat.shape, lambda n: (0, 0, 0)),
            pl.BlockSpec(c1_col.shape, lambda n: (0, 0)),
            pl.BlockSpec(wtf.shape, lambda n: (0, 0)),
            pl.BlockSpec(c3_col.shape, lambda n: (0, 0)),
        ],
        out_specs=pl.BlockSpec((None, O, TV), lambda n: (n, 0, 0)),
        scratch_shapes=[
            pltpu.VMEM((KT * C, TV), jnp.bfloat16),
            pltpu.VMEM((r1 + r9, TV), jnp.bfloat16),
            pltpu.VMEM((pg_rows, L4), jnp.bfloat16),
            pltpu.VMEM((O * G, NS * L4), jnp.bfloat16),
            pltpu.VMEM((O, TV), jnp.bfloat16),
            pltpu.VMEM((KT * O, TV), jnp.bfloat16),
        ],
        compiler_params=pltpu.CompilerParams(dimension_semantics=("parallel",),
                                             vmem_limit_bytes=VMEM_LIMIT),
    )(xb, w1, b1, w9, b9, acat, c1_col, wtf, c3_col)
    return out.reshape(N, O, T, V)
